# baseline (speedup 1.0000x reference)
import functools

import jax
import jax.numpy as jnp
import numpy as np
from jax import lax
from jax.experimental import pallas as pl
from jax.experimental.pallas import tpu as pltpu
from jax.experimental.pallas import tpu_sc as plsc


@functools.lru_cache(maxsize=None)
def _sc_kernel(C, CS, T, H, W, n_slow, hch, nw):
    mesh = plsc.VectorSubcoreMesh(core_axis_name="c", subcore_axis_name="s")
    slow_units = C * n_slow * hch
    fast_units = CS * T * hch
    slow_pw = slow_units // nw
    fast_pw = fast_units // nw
    per_w = slow_pw + fast_pw
    hblk = H // hch
    nbuf = 2

    @functools.partial(
        pl.kernel,
        out_type=(
            jax.ShapeDtypeStruct((C, n_slow, H, W), jnp.float32),
            jax.ShapeDtypeStruct((CS, T, H, W), jnp.float32),
        ),
        mesh=mesh,
        scratch_types=[
            pltpu.VMEM((nbuf, hblk, W), jnp.float32),
            pltpu.SemaphoreType.DMA,
            pltpu.SemaphoreType.DMA,
        ],
    )
    def k(in_hbm, slow_hbm, fast_hbm, buf, sem_r, sem_w):
        wid = lax.axis_index("s") * 2 + lax.axis_index("c")

        def unit(i):
            if i < slow_pw:
                j = wid * slow_pw + i
                part = lax.rem(j, hch)
                r = lax.div(j, hch)
                c = lax.div(r, n_slow)
                t = lax.rem(r, n_slow)
                src_t = lax.div(t * (T - 1), n_slow - 1)
                h0 = part * hblk
                return (c, src_t, h0), slow_hbm, (c, t, h0)
            j = wid * fast_pw + (i - slow_pw)
            part = lax.rem(j, hch)
            r = lax.div(j, hch)
            c = lax.div(r, T)
            t = lax.rem(r, T)
            h0 = part * hblk
            return (c, t, h0), fast_hbm, (c, t, h0)

        us = [unit(i) for i in range(per_w)]

        def rd_cp(i):
            (c, t, h0), _, _ = us[i]
            return pltpu.async_copy(
                in_hbm.at[c, t, pl.ds(h0, hblk)], buf.at[i % nbuf], sem_r)

        def wr_cp(i):
            _, dst, (c, t, h0) = us[i]
            return pltpu.async_copy(
                buf.at[i % nbuf], dst.at[c, t, pl.ds(h0, hblk)], sem_w)

        rd = [rd_cp(i) for i in range(min(nbuf, per_w))]
        wr = [None] * per_w
        for i in range(per_w):
            rd[i].wait()
            wr[i] = wr_cp(i)
            if i + nbuf < per_w:
                wr[i].wait()
                rd.append(rd_cp(i + nbuf))
        for i in range(max(0, per_w - nbuf), per_w):
            wr[i].wait()

    return k


@functools.lru_cache(maxsize=None)
def _fast_rest_kernel(C, CS, T, H, W, nbuf, grp, tsub):
    CR = C - CS
    TS = T // tsub
    N = CR * TS

    def body(in_ref, out_ref, buf, sem_in, sem_out):
        def cp_in(u):
            return pltpu.make_async_copy(
                in_ref.at[CS + u // TS, pl.ds((u % TS) * tsub, tsub)],
                buf.at[u % nbuf], sem_in.at[u % nbuf])

        def cp_out(u):
            return pltpu.make_async_copy(
                buf.at[u % nbuf],
                out_ref.at[u // TS, pl.ds((u % TS) * tsub, tsub)],
                sem_out.at[u % nbuf])

        ngrp = N // grp
        for j in range(grp):
            cp_in(j).start()
        for k in range(ngrp):
            if k + 1 < ngrp:
                for j in range(grp):
                    cp_in((k + 1) * grp + j).start()
            for j in range(grp):
                u = k * grp + j
                cp_in(u).wait()
                cp_out(u).start()
            for j in range(grp):
                cp_out(k * grp + j).wait()

    return pl.pallas_call(
        body,
        in_specs=[pl.BlockSpec(memory_space=pl.ANY)],
        out_specs=pl.BlockSpec(memory_space=pl.ANY),
        out_shape=jax.ShapeDtypeStruct((CR, T, H, W), jnp.float32),
        scratch_shapes=[
            pltpu.VMEM((nbuf, tsub, H, W), jnp.float32),
            pltpu.SemaphoreType.DMA((nbuf,)),
            pltpu.SemaphoreType.DMA((nbuf,)),
        ],
    )


def kernel(frames):
    C, T, H, W = frames.shape
    alpha = 4
    n_slow = T // alpha
    idx_ref = np.linspace(0, T - 1, n_slow).astype(np.int64)
    idx_arith = (np.arange(n_slow) * (T - 1)) // (n_slow - 1)
    assert (idx_ref == idx_arith).all()

    info = plsc.get_sparse_core_info()
    nw = info.num_cores * info.num_subcores
    CS = 1

    hch = 1
    while ((C * n_slow * hch) % nw != 0 or (CS * T * hch) % nw != 0
           or H % hch != 0 or 2 * (H // hch) * W * 4 > 524284):
        hch *= 2

    slow, fast0 = _sc_kernel(C, CS, T, H, W, n_slow, hch, nw)(frames)
    fast_rest = _fast_rest_kernel(C, CS, T, H, W, 4, 2, 16)(frames)
    fast = jnp.concatenate([fast0, fast_rest], axis=0)
    return (slow, fast)


# speedup vs baseline: 1.6895x; 1.6895x over previous
import functools

import jax
import jax.numpy as jnp
import numpy as np
from jax import lax
from jax.experimental import pallas as pl
from jax.experimental.pallas import tpu as pltpu
from jax.experimental.pallas import tpu_sc as plsc


@functools.lru_cache(maxsize=None)
def _slow_gather_kernel(C, T, H, W, n_slow, hch, nw):
    mesh = plsc.VectorSubcoreMesh(core_axis_name="c", subcore_axis_name="s")
    units = C * n_slow * hch
    per_w = units // nw
    hblk = H // hch
    nbuf = 2

    @functools.partial(
        pl.kernel,
        out_type=jax.ShapeDtypeStruct((C, n_slow, H, W), jnp.float32),
        mesh=mesh,
        scratch_types=[
            pltpu.VMEM((nbuf, hblk, W), jnp.float32),
            pltpu.SemaphoreType.DMA,
            pltpu.SemaphoreType.DMA,
        ],
    )
    def k(in_hbm, out_hbm, buf, sem_r, sem_w):
        wid = lax.axis_index("s") * 2 + lax.axis_index("c")

        def unit(i):
            j = wid * per_w + i
            part = lax.rem(j, hch)
            r = lax.div(j, hch)
            c = lax.div(r, n_slow)
            t = lax.rem(r, n_slow)
            src_t = lax.div(t * (T - 1), n_slow - 1)
            h0 = part * hblk
            return c, src_t, t, h0

        us = [unit(i) for i in range(per_w)]
        rd = [pltpu.async_copy(
                  in_hbm.at[us[i][0], us[i][1], pl.ds(us[i][3], hblk)],
                  buf.at[i], sem_r)
              for i in range(min(nbuf, per_w))]
        wr = [None] * per_w
        for i in range(per_w):
            c, src_t, t, h0 = us[i]
            rd[i].wait()
            wr[i] = pltpu.async_copy(
                buf.at[i % nbuf], out_hbm.at[c, t, pl.ds(h0, hblk)], sem_w)
            if i + nbuf < per_w:
                wr[i].wait()
                cn, srcn, tn, h0n = us[i + nbuf]
                rd.append(pltpu.async_copy(
                    in_hbm.at[cn, srcn, pl.ds(h0n, hblk)],
                    buf.at[i % nbuf], sem_r))
        for i in range(max(0, per_w - nbuf), per_w):
            wr[i].wait()

    return k


@functools.lru_cache(maxsize=None)
def _fast_copy_kernel(C, T, H, W, nbuf, grp, tsub):
    TS = T // tsub
    N = C * TS

    def body(in_ref, out_ref, buf, sem_in, sem_out):
        def cp_in(u):
            return pltpu.make_async_copy(
                in_ref.at[u // TS, pl.ds((u % TS) * tsub, tsub)],
                buf.at[u % nbuf], sem_in.at[u % nbuf])

        def cp_out(u):
            return pltpu.make_async_copy(
                buf.at[u % nbuf],
                out_ref.at[u // TS, pl.ds((u % TS) * tsub, tsub)],
                sem_out.at[u % nbuf])

        ngrp = N // grp
        for j in range(grp):
            cp_in(j).start()
        for k in range(ngrp):
            if k + 1 < ngrp:
                for j in range(grp):
                    cp_in((k + 1) * grp + j).start()
            for j in range(grp):
                u = k * grp + j
                cp_in(u).wait()
                cp_out(u).start()
            for j in range(grp):
                cp_out(k * grp + j).wait()

    return pl.pallas_call(
        body,
        in_specs=[pl.BlockSpec(memory_space=pl.ANY)],
        out_specs=pl.BlockSpec(memory_space=pl.ANY),
        out_shape=jax.ShapeDtypeStruct((C, T, H, W), jnp.float32),
        scratch_shapes=[
            pltpu.VMEM((nbuf, tsub, H, W), jnp.float32),
            pltpu.SemaphoreType.DMA((nbuf,)),
            pltpu.SemaphoreType.DMA((nbuf,)),
        ],
    )


def kernel(frames):
    C, T, H, W = frames.shape
    alpha = 4
    n_slow = T // alpha
    idx_ref = np.linspace(0, T - 1, n_slow).astype(np.int64)
    idx_arith = (np.arange(n_slow) * (T - 1)) // (n_slow - 1)
    assert (idx_ref == idx_arith).all()

    info = plsc.get_sparse_core_info()
    nw = info.num_cores * info.num_subcores

    hch = 1
    while ((C * n_slow * hch) % nw != 0 or H % hch != 0
           or 2 * (H // hch) * W * 4 > 524284):
        hch *= 2

    slow = _slow_gather_kernel(C, T, H, W, n_slow, hch, nw)(frames)
    fast = _fast_copy_kernel(C, T, H, W, 8, 4, 8)(frames)
    return (slow, fast)


# speedup vs baseline: 1.6920x; 1.0015x over previous
import functools

import jax
import jax.numpy as jnp
import numpy as np
from jax import lax
from jax.experimental import pallas as pl
from jax.experimental.pallas import tpu as pltpu
from jax.experimental.pallas import tpu_sc as plsc


@functools.lru_cache(maxsize=None)
def _slow_gather_kernel(C, T, H, W, n_slow, hch, nw):
    mesh = plsc.VectorSubcoreMesh(core_axis_name="c", subcore_axis_name="s")
    units = C * n_slow * hch
    per_w = units // nw
    hblk = H // hch
    nbuf = 2

    @functools.partial(
        pl.kernel,
        out_type=jax.ShapeDtypeStruct((C, n_slow, H, W), jnp.float32),
        mesh=mesh,
        scratch_types=[
            pltpu.VMEM((nbuf, hblk, W), jnp.float32),
            pltpu.SemaphoreType.DMA,
            pltpu.SemaphoreType.DMA,
        ],
    )
    def k(in_hbm, out_hbm, buf, sem_r, sem_w):
        wid = lax.axis_index("s") * 2 + lax.axis_index("c")

        def unit(i):
            j = wid * per_w + i
            part = lax.rem(j, hch)
            r = lax.div(j, hch)
            c = lax.div(r, n_slow)
            t = lax.rem(r, n_slow)
            src_t = lax.div(t * (T - 1), n_slow - 1)
            h0 = part * hblk
            return c, src_t, t, h0

        us = [unit(i) for i in range(per_w)]
        rd = [pltpu.async_copy(
                  in_hbm.at[us[i][0], us[i][1], pl.ds(us[i][3], hblk)],
                  buf.at[i], sem_r)
              for i in range(min(nbuf, per_w))]
        wr = [None] * per_w
        for i in range(per_w):
            c, src_t, t, h0 = us[i]
            rd[i].wait()
            wr[i] = pltpu.async_copy(
                buf.at[i % nbuf], out_hbm.at[c, t, pl.ds(h0, hblk)], sem_w)
            if i + nbuf < per_w:
                wr[i].wait()
                cn, srcn, tn, h0n = us[i + nbuf]
                rd.append(pltpu.async_copy(
                    in_hbm.at[cn, srcn, pl.ds(h0n, hblk)],
                    buf.at[i % nbuf], sem_r))
        for i in range(max(0, per_w - nbuf), per_w):
            wr[i].wait()

    return k


@functools.lru_cache(maxsize=None)
def _fast_copy_kernel(C, T, H, W, nbuf, grp, tsub):
    TS = T // tsub
    N = C * TS

    def body(in_ref, out_ref, buf, sem_in, sem_out):
        def cp_in(u):
            return pltpu.make_async_copy(
                in_ref.at[u // TS, pl.ds((u % TS) * tsub, tsub)],
                buf.at[u % nbuf], sem_in.at[u % nbuf])

        def cp_out(u):
            return pltpu.make_async_copy(
                buf.at[u % nbuf],
                out_ref.at[u // TS, pl.ds((u % TS) * tsub, tsub)],
                sem_out.at[u % nbuf])

        ngrp = N // grp
        for j in range(grp):
            cp_in(j).start()
        for k in range(ngrp):
            if k + 1 < ngrp:
                for j in range(grp):
                    cp_in((k + 1) * grp + j).start()
            for j in range(grp):
                u = k * grp + j
                cp_in(u).wait()
                cp_out(u).start()
            for j in range(grp):
                cp_out(k * grp + j).wait()

    return pl.pallas_call(
        body,
        in_specs=[pl.BlockSpec(memory_space=pl.ANY)],
        out_specs=pl.BlockSpec(memory_space=pl.ANY),
        out_shape=jax.ShapeDtypeStruct((C, T, H, W), jnp.float32),
        scratch_shapes=[
            pltpu.VMEM((nbuf, tsub, H, W), jnp.float32),
            pltpu.SemaphoreType.DMA((nbuf,)),
            pltpu.SemaphoreType.DMA((nbuf,)),
        ],
    )


def kernel(frames):
    C, T, H, W = frames.shape
    alpha = 4
    n_slow = T // alpha
    idx_ref = np.linspace(0, T - 1, n_slow).astype(np.int64)
    idx_arith = (np.arange(n_slow) * (T - 1)) // (n_slow - 1)
    assert (idx_ref == idx_arith).all()

    info = plsc.get_sparse_core_info()
    nw = info.num_cores * info.num_subcores

    hch = 1
    while ((C * n_slow * hch) % nw != 0 or H % hch != 0
           or 2 * (H // hch) * W * 4 > 524284):
        hch *= 2

    slow = _slow_gather_kernel(C, T, H, W, n_slow, hch, nw)(frames)
    fast = _fast_copy_kernel(C, T, H, W, 12, 6, 8)(frames)
    return (slow, fast)
